# baseline (speedup 1.0000x reference)
import dataclasses
import functools

import jax
import jax.numpy as jnp
from jax import lax
from jax.experimental import pallas as pl
from jax.experimental.pallas import tpu as pltpu
from jax.experimental.pallas import tpu_sc as plsc

B = 128
N = 64
EPG = 512
IN = 128
HID = 64
LAT = 32
E = B * EPG
GPW = 16
NW = B // GPW
EPW = GPW * EPG
WPW = GPW * N * N


def _adj_sc(rows, cols, zeros_hbm):
    mesh = plsc.VectorSubcoreMesh(core_axis_name="c", subcore_axis_name="s")
    cp = pltpu.CompilerParams()
    if "needs_layout_passes" in pltpu.CompilerParams.__dataclass_fields__:
        cp = dataclasses.replace(cp, needs_layout_passes=False)

    @functools.partial(
        pl.kernel,
        compiler_params=cp,
        out_type=[
            jax.ShapeDtypeStruct((B * N * N,), jnp.float32),
            jax.ShapeDtypeStruct((B * N * N,), jnp.float32),
            jax.ShapeDtypeStruct((B * N,), jnp.float32),
        ],
        mesh=mesh,
        scratch_types=[
            pltpu.VMEM((EPW,), jnp.int32),
            pltpu.VMEM((EPW,), jnp.int32),
            pltpu.VMEM((WPW,), jnp.float32),
            pltpu.VMEM((GPW * N,), jnp.float32),
        ],
    )
    def k(rows_hbm, cols_hbm, zeros_hbm, adj_out, adjt_out, deg_out,
          rv, cv, counts, degc):
        wid = lax.axis_index("s") * 2 + lax.axis_index("c")

        @pl.when(wid < 2 * NW)
        def _():
            gw = wid % NW
            is_adj = wid < NW
            ebase = gw * EPW
            pltpu.sync_copy(zeros_hbm, counts)
            pltpu.sync_copy(zeros_hbm.at[pl.ds(0, GPW * N)], degc)
            pltpu.sync_copy(rows_hbm.at[pl.ds(ebase, EPW)], rv)
            pltpu.sync_copy(cols_hbm.at[pl.ds(ebase, EPW)], cv)

            iota16 = lax.iota(jnp.int32, 16)
            lane_base = iota16 * (N * N)
            gather_base = iota16 * EPG
            ones16 = jnp.ones((16,), jnp.float32)
            degmask = jnp.broadcast_to(is_adj, (16,))
            UNROLL = 4

            def edge_step(i, c):
                for u in range(UNROLL):
                    e = i * UNROLL + u
                    r = plsc.load_gather(rv, [gather_base + e])
                    cc = plsc.load_gather(cv, [gather_base + e])
                    rloc = r & (N - 1)
                    cloc = cc & (N - 1)
                    fi = jnp.where(is_adj, rloc * N + cloc, cloc * N + rloc)
                    plsc.addupdate_scatter(counts, [lane_base + fi], ones16)
                    plsc.addupdate_scatter(degc, [iota16 * N + cloc], ones16,
                                           mask=degmask)
                return c

            lax.fori_loop(0, EPG // UNROLL, edge_step, 0)

            obase = gw * WPW

            @pl.when(is_adj)
            def _():
                pltpu.sync_copy(counts, adj_out.at[pl.ds(obase, WPW)])
                pltpu.sync_copy(degc, deg_out.at[pl.ds(gw * GPW * N, GPW * N)])

            @pl.when(jnp.logical_not(is_adj))
            def _():
                pltpu.sync_copy(counts, adjt_out.at[pl.ds(obase, WPW)])

    return k(rows, cols, zeros_hbm)


def _dense_body(x_ref, adjt_ref, degc_ref, eps_ref,
                w1_ref, b1_ref, g1_ref, be1_ref,
                w2_ref, b2_ref, g2_ref, be2_ref,
                w3_ref, b3_ref, g3_ref, be3_ref,
                muw_ref, mub_ref, lvw_ref, lvb_ref,
                d1w_ref, d1b_ref, d2w_ref, d2b_ref,
                pw_ref, pb_ref, e1a_ref, e1b_ref, e1bias_ref,
                e2v_ref, e2b_ref,
                logits_ref, mu_ref, lv_ref,
                a1_s, a2_s):
    x = x_ref[...]
    dinv = lax.rsqrt(degc_ref[...] + 1.0)

    row_i = lax.broadcasted_iota(jnp.int32, (N, N), 0)
    col_i = lax.broadcasted_iota(jnp.int32, (N, N), 1)
    eye = jnp.where(row_i == col_i, 1.0, 0.0).astype(jnp.float32)
    at_aug = adjt_ref[...] + eye[None, :, :]

    def gcn_bn_layer(h, w_ref, b_ref, g_ref, be_ref):
        hw = jnp.dot(h, w_ref[...]) * dinv
        hw3 = hw.reshape(B, N, HID)
        agg3 = lax.dot_general(at_aug, hw3,
                               (((2,), (1,)), ((0,), (0,))))
        agg = agg3.reshape(B * N, HID) * dinv + b_ref[...]
        m = jnp.mean(agg, axis=0, keepdims=True)
        msq = jnp.mean(agg * agg, axis=0, keepdims=True)
        v = msq - m * m
        hn = (agg - m) * lax.rsqrt(v + 1e-5) * g_ref[...] + be_ref[...]
        return jnp.maximum(hn, 0.2 * hn)

    h = gcn_bn_layer(x, w1_ref, b1_ref, g1_ref, be1_ref)
    h = gcn_bn_layer(h, w2_ref, b2_ref, g2_ref, be2_ref)
    h = gcn_bn_layer(h, w3_ref, b3_ref, g3_ref, be3_ref)

    g = h.reshape(B, N, HID).sum(axis=1) * (1.0 / N)
    mu = jnp.dot(g, muw_ref[...]) + mub_ref[...]
    lv = jnp.dot(g, lvw_ref[...]) + lvb_ref[...]
    mu_ref[...] = mu
    lv_ref[...] = lv
    z = mu + jnp.exp(0.5 * lv) * eps_ref[...]
    hg = jnp.dot(z, d1w_ref[...]) + d1b_ref[...]
    hg = jnp.maximum(hg, 0.0)
    hg = jnp.dot(hg, d2w_ref[...]) + d2b_ref[...]

    hn = (jnp.dot(x, pw_ref[...]) + pb_ref[...]).reshape(B, N, HID)
    hn = hn + hg[:, None, :]
    hnf = hn.reshape(B * N, HID)
    a1_s[...] = (jnp.dot(hnf, e1a_ref[...])
                 + e1bias_ref[...]).reshape(B, N, HID).astype(jnp.bfloat16)
    a2_s[...] = jnp.dot(hnf, e1b_ref[...]).reshape(B, N, HID).astype(jnp.bfloat16)

    e2v = e2v_ref[...].astype(jnp.bfloat16)
    e2b = e2b_ref[0, 0]
    GB = 8

    def blk(gb, c):
        a1b = a1_s[pl.ds(gb * GB, GB)]
        a2b = a2_s[pl.ds(gb * GB, GB)]
        p = a1b[:, :, None, :] + a2b[:, None, :, :]
        r = jnp.maximum(p, 0)
        s = jnp.sum(r * e2v, axis=-1).astype(jnp.float32) + e2b
        logits_ref[pl.ds(gb * GB, GB)] = s
        return c

    lax.fori_loop(0, B // GB, blk, 0)


def kernel(x, params, edge_index, batch):
    p = params
    rows = edge_index[0]
    cols = edge_index[1]

    zeros_hbm = jnp.zeros((WPW,), jnp.float32)
    adj_flat, adjt_flat, deg_cnt = _adj_sc(rows, cols, zeros_hbm)
    adj_gt = adj_flat.reshape(B, N, N)
    adjt = adjt_flat.reshape(B, N, N)

    eps = jax.random.normal(jax.random.key(42), (B, LAT), dtype=jnp.float32)

    def r2(a):
        return a.reshape(1, -1)

    logits, mu, logvar = pl.pallas_call(
        _dense_body,
        out_shape=[
            jax.ShapeDtypeStruct((B, N, N), jnp.float32),
            jax.ShapeDtypeStruct((B, LAT), jnp.float32),
            jax.ShapeDtypeStruct((B, LAT), jnp.float32),
        ],
        in_specs=[pl.BlockSpec(memory_space=pltpu.VMEM)] * 30 + [
            pl.BlockSpec(memory_space=pltpu.SMEM)],
        out_specs=[pl.BlockSpec(memory_space=pltpu.VMEM)] * 3,
        scratch_shapes=[
            pltpu.VMEM((B, N, HID), jnp.bfloat16),
            pltpu.VMEM((B, N, HID), jnp.bfloat16),
        ],
    )(
        x, adjt, deg_cnt.reshape(B * N, 1), eps,
        p['conv1_w'], r2(p['conv1_b']), r2(p['bn1_g']), r2(p['bn1_b']),
        p['conv2_w'], r2(p['conv2_b']), r2(p['bn2_g']), r2(p['bn2_b']),
        p['conv3_w'], r2(p['conv3_b']), r2(p['bn3_g']), r2(p['bn3_b']),
        p['mu_w'], r2(p['mu_b']), p['lv_w'], r2(p['lv_b']),
        p['dec1_w'], r2(p['dec1_b']), p['dec2_w'], r2(p['dec2_b']),
        p['proj_w'], r2(p['proj_b']),
        p['ed1_w'][:HID], p['ed1_w'][HID:], r2(p['ed1_b']),
        p['ed2_w'].reshape(1, HID), p['ed2_b'].reshape(1, 1),
    )

    mask = jnp.ones((B, N), dtype=bool)
    return logits, adj_gt, mask, mu, logvar


# speedup vs baseline: 1.0467x; 1.0467x over previous
import dataclasses
import functools

import jax
import jax.numpy as jnp
from jax import lax
from jax.experimental import pallas as pl
from jax.experimental.pallas import tpu as pltpu
from jax.experimental.pallas import tpu_sc as plsc

B = 128
N = 64
EPG = 512
IN = 128
HID = 64
LAT = 32
E = B * EPG
GPW = 16
NW = B // GPW
EPW = GPW * EPG
WPW = GPW * N * N


def _adj_sc(rows, cols, zeros_hbm):
    mesh = plsc.VectorSubcoreMesh(core_axis_name="c", subcore_axis_name="s")
    cp = pltpu.CompilerParams()
    if "needs_layout_passes" in pltpu.CompilerParams.__dataclass_fields__:
        cp = dataclasses.replace(cp, needs_layout_passes=False)

    @functools.partial(
        pl.kernel,
        compiler_params=cp,
        out_type=[
            jax.ShapeDtypeStruct((B * N * N,), jnp.float32),
            jax.ShapeDtypeStruct((B * N * N,), jnp.float32),
            jax.ShapeDtypeStruct((B * N,), jnp.float32),
        ],
        mesh=mesh,
        scratch_types=[
            pltpu.VMEM((EPW,), jnp.int32),
            pltpu.VMEM((EPW,), jnp.int32),
            pltpu.VMEM((WPW,), jnp.float32),
            pltpu.VMEM((GPW * N,), jnp.float32),
            pltpu.SemaphoreType.DMA,
        ],
    )
    def k(rows_hbm, cols_hbm, zeros_hbm, adj_out, adjt_out, deg_out,
          rv, cv, counts, degc, sem):
        wid = lax.axis_index("s") * 2 + lax.axis_index("c")

        @pl.when(wid < 2 * NW)
        def _():
            gw = wid % NW
            is_adj = wid < NW
            ebase = gw * EPW
            cps = [pltpu.async_copy(zeros_hbm, counts, sem),
                   pltpu.async_copy(zeros_hbm.at[pl.ds(0, GPW * N)], degc, sem),
                   pltpu.async_copy(rows_hbm.at[pl.ds(ebase, EPW)], rv, sem),
                   pltpu.async_copy(cols_hbm.at[pl.ds(ebase, EPW)], cv, sem)]
            for cp in cps:
                cp.wait()

            iota16 = lax.iota(jnp.int32, 16)
            lane_base = iota16 * (N * N)
            gather_base = iota16 * EPG
            ones16 = jnp.ones((16,), jnp.float32)
            degmask = jnp.broadcast_to(is_adj, (16,))
            UNROLL = 8

            def edge_step(i, c):
                for u in range(UNROLL):
                    e = i * UNROLL + u
                    r = plsc.load_gather(rv, [gather_base + e])
                    cc = plsc.load_gather(cv, [gather_base + e])
                    rloc = r & (N - 1)
                    cloc = cc & (N - 1)
                    fi = jnp.where(is_adj, rloc * N + cloc, cloc * N + rloc)
                    plsc.addupdate_scatter(counts, [lane_base + fi], ones16)
                    plsc.addupdate_scatter(degc, [iota16 * N + cloc], ones16,
                                           mask=degmask)
                return c

            lax.fori_loop(0, EPG // UNROLL, edge_step, 0)

            obase = gw * WPW

            @pl.when(is_adj)
            def _():
                pltpu.sync_copy(counts, adj_out.at[pl.ds(obase, WPW)])
                pltpu.sync_copy(degc, deg_out.at[pl.ds(gw * GPW * N, GPW * N)])

            @pl.when(jnp.logical_not(is_adj))
            def _():
                pltpu.sync_copy(counts, adjt_out.at[pl.ds(obase, WPW)])

    return k(rows, cols, zeros_hbm)


def _dense_body(x_ref, adjt_ref, degc_ref, eps_ref,
                w1_ref, b1_ref, g1_ref, be1_ref,
                w2_ref, b2_ref, g2_ref, be2_ref,
                w3_ref, b3_ref, g3_ref, be3_ref,
                muw_ref, mub_ref, lvw_ref, lvb_ref,
                d1w_ref, d1b_ref, d2w_ref, d2b_ref,
                pw_ref, pb_ref, e1a_ref, e1b_ref, e1bias_ref,
                e2v_ref, e2b_ref,
                logits_ref, mu_ref, lv_ref,
                a1_s, a2_s):
    x = x_ref[...]
    dinv = lax.rsqrt(degc_ref[...] + 1.0)

    row_i = lax.broadcasted_iota(jnp.int32, (N, N), 0)
    col_i = lax.broadcasted_iota(jnp.int32, (N, N), 1)
    eye = jnp.where(row_i == col_i, 1.0, 0.0).astype(jnp.float32)
    at_aug = adjt_ref[...] + eye[None, :, :]

    def gcn_bn_layer(h, w_ref, b_ref, g_ref, be_ref):
        hw = jnp.dot(h, w_ref[...]) * dinv
        hw3 = hw.reshape(B, N, HID)
        agg3 = lax.dot_general(at_aug, hw3,
                               (((2,), (1,)), ((0,), (0,))))
        agg = agg3.reshape(B * N, HID) * dinv + b_ref[...]
        m = jnp.mean(agg, axis=0, keepdims=True)
        msq = jnp.mean(agg * agg, axis=0, keepdims=True)
        v = msq - m * m
        hn = (agg - m) * lax.rsqrt(v + 1e-5) * g_ref[...] + be_ref[...]
        return jnp.maximum(hn, 0.2 * hn)

    h = gcn_bn_layer(x, w1_ref, b1_ref, g1_ref, be1_ref)
    h = gcn_bn_layer(h, w2_ref, b2_ref, g2_ref, be2_ref)
    h = gcn_bn_layer(h, w3_ref, b3_ref, g3_ref, be3_ref)

    g = h.reshape(B, N, HID).sum(axis=1) * (1.0 / N)
    mu = jnp.dot(g, muw_ref[...]) + mub_ref[...]
    lv = jnp.dot(g, lvw_ref[...]) + lvb_ref[...]
    mu_ref[...] = mu
    lv_ref[...] = lv
    z = mu + jnp.exp(0.5 * lv) * eps_ref[...]
    hg = jnp.dot(z, d1w_ref[...]) + d1b_ref[...]
    hg = jnp.maximum(hg, 0.0)
    hg = jnp.dot(hg, d2w_ref[...]) + d2b_ref[...]

    hn = (jnp.dot(x, pw_ref[...]) + pb_ref[...]).reshape(B, N, HID)
    hn = hn + hg[:, None, :]
    hnf = hn.reshape(B * N, HID)
    a1_s[...] = (jnp.dot(hnf, e1a_ref[...]) + e1bias_ref[...]).reshape(B, N, HID)
    a2_s[...] = jnp.dot(hnf, e1b_ref[...]).reshape(B, N, HID)

    e2v = e2v_ref[...]
    e2b = e2b_ref[0, 0]
    GB = 8

    def blk(gb, c):
        a1b = a1_s[pl.ds(gb * GB, GB)]
        a2b = a2_s[pl.ds(gb * GB, GB)]
        p = a1b[:, :, None, :] + a2b[:, None, :, :]
        r = jnp.maximum(p, 0.0)
        s = jnp.sum(r * e2v, axis=-1) + e2b
        logits_ref[pl.ds(gb * GB, GB)] = s
        return c

    lax.fori_loop(0, B // GB, blk, 0)


def kernel(x, params, edge_index, batch):
    p = params
    rows = edge_index[0]
    cols = edge_index[1]

    zeros_hbm = jnp.zeros((WPW,), jnp.float32)
    adj_flat, adjt_flat, deg_cnt = _adj_sc(rows, cols, zeros_hbm)
    adj_gt = adj_flat.reshape(B, N, N)
    adjt = adjt_flat.reshape(B, N, N)

    eps = jax.random.normal(jax.random.key(42), (B, LAT), dtype=jnp.float32)

    def r2(a):
        return a.reshape(1, -1)

    logits, mu, logvar = pl.pallas_call(
        _dense_body,
        out_shape=[
            jax.ShapeDtypeStruct((B, N, N), jnp.float32),
            jax.ShapeDtypeStruct((B, LAT), jnp.float32),
            jax.ShapeDtypeStruct((B, LAT), jnp.float32),
        ],
        in_specs=[pl.BlockSpec(memory_space=pltpu.VMEM)] * 30 + [
            pl.BlockSpec(memory_space=pltpu.SMEM)],
        out_specs=[pl.BlockSpec(memory_space=pltpu.VMEM)] * 3,
        scratch_shapes=[
            pltpu.VMEM((B, N, HID), jnp.float32),
            pltpu.VMEM((B, N, HID), jnp.float32),
        ],
    )(
        x, adjt, deg_cnt.reshape(B * N, 1), eps,
        p['conv1_w'], r2(p['conv1_b']), r2(p['bn1_g']), r2(p['bn1_b']),
        p['conv2_w'], r2(p['conv2_b']), r2(p['bn2_g']), r2(p['bn2_b']),
        p['conv3_w'], r2(p['conv3_b']), r2(p['bn3_g']), r2(p['bn3_b']),
        p['mu_w'], r2(p['mu_b']), p['lv_w'], r2(p['lv_b']),
        p['dec1_w'], r2(p['dec1_b']), p['dec2_w'], r2(p['dec2_b']),
        p['proj_w'], r2(p['proj_b']),
        p['ed1_w'][:HID], p['ed1_w'][HID:], r2(p['ed1_b']),
        p['ed2_w'].reshape(1, HID), p['ed2_b'].reshape(1, 1),
    )

    mask = jnp.ones((B, N), dtype=bool)
    return logits, adj_gt, mask, mu, logvar
